# baseline (speedup 1.0000x reference)
import functools

import jax
import jax.numpy as jnp
from jax.experimental import pallas as pl
from jax.experimental.pallas import tpu as pltpu

_HIGH = jax.lax.Precision.HIGHEST
_NL = 8
_TM = 512


def _top2_coeff(z, tau):
    idx = jax.lax.broadcasted_iota(jnp.int32, z.shape, 1)
    v1 = jnp.max(z, axis=1, keepdims=True)
    i1 = jnp.min(jnp.where(z == v1, idx, z.shape[1]), axis=1, keepdims=True)
    m1 = idx == i1
    z2 = jnp.where(m1, -jnp.inf, z)
    v2 = jnp.max(z2, axis=1, keepdims=True)
    i2 = jnp.min(jnp.where(z2 == v2, idx, z.shape[1]), axis=1, keepdims=True)
    m2 = idx == i2
    t = tau + 1e-8
    a = jnp.exp((v2 - v1) / t)
    w1 = 1.0 / (1.0 + a)
    w2 = a / (1.0 + a)
    return jnp.where(m1, w1, 0.0) + jnp.where(m2, w2, 0.0)


def _col(c, l):
    oh = (jax.lax.broadcasted_iota(jnp.int32, (_NL, 1), 0) == l)
    return jax.lax.dot_general(c, oh.astype(jnp.float32),
                               (((1,), (0,)), ((), ())),
                               precision=_HIGH,
                               preferred_element_type=jnp.float32)


def _layer1_body(x_ref, pw_ref, u_ref, w_ref, hb_ref, cc_ref,
                 acc_ref, wb_ref, c1_ref):
    t = pl.program_id(0)
    l = pl.program_id(1)

    @pl.when(t == 0)
    def _cast():
        wb_ref[pl.ds(l, 1)] = w_ref[...].astype(jnp.bfloat16)

    xb = x_ref[...].astype(jnp.bfloat16)

    @pl.when(l == 0)
    def _routing():
        addr = jax.lax.dot_general(xb, pw_ref[...], (((1,), (0,)), ((), ())),
                                   preferred_element_type=jnp.float32)
        zz = jax.lax.dot_general(addr.astype(jnp.bfloat16), u_ref[...],
                                 (((1,), (0,)), ((), ())),
                                 preferred_element_type=jnp.float32)
        c1_ref[...] = _top2_coeff(zz[:, 0:8], 1.0)
        cc_ref[...] = jnp.concatenate(
            [_top2_coeff(zz[:, 8:16], 1.0), _top2_coeff(zz[:, 16:24], 1.0)],
            axis=1)

    yl = jax.lax.dot_general(xb, wb_ref[l], (((1,), (1,)), ((), ())),
                             preferred_element_type=jnp.float32)
    contrib = yl * _col(c1_ref[...], l)

    @pl.when(l == 0)
    def _init():
        acc_ref[...] = contrib

    @pl.when(l > 0)
    def _accum():
        acc_ref[...] = acc_ref[...] + contrib

    @pl.when(l == _NL - 1)
    def _fin():
        h = acc_ref[...]
        h = 0.5 * h * (1.0 + jax.lax.erf(h * 0.7071067811865476))
        hb_ref[...] = h.astype(jnp.bfloat16)


def _layer2_body(hb_ref, cc_ref, b2_ref, w_ref, y_ref, acc_ref, wb_ref):
    t = pl.program_id(0)
    l = pl.program_id(1)

    @pl.when(t == 0)
    def _cast():
        wb_ref[pl.ds(l, 1)] = w_ref[...].astype(jnp.bfloat16)

    yl = jax.lax.dot_general(hb_ref[...], wb_ref[l], (((1,), (1,)), ((), ())),
                             preferred_element_type=jnp.float32)
    contrib = yl * _col(cc_ref[:, 0:8], l)

    @pl.when(l == 0)
    def _init():
        bias = jax.lax.dot_general(cc_ref[:, 8:16], b2_ref[...],
                                   (((1,), (0,)), ((), ())),
                                   precision=_HIGH,
                                   preferred_element_type=jnp.float32)
        acc_ref[...] = bias + contrib

    @pl.when(l > 0)
    def _accum():
        acc_ref[...] = acc_ref[...] + contrib

    @pl.when(l == _NL - 1)
    def _fin():
        y_ref[...] = acc_ref[...]


@functools.partial(jax.jit, static_argnames=())
def kernel(x, P_w, U1, U2, U3, W1, W2, b2):
    Bx, Tx, D = x.shape
    N = Bx * Tx
    DO = W2.shape[1]
    x_flat = x.reshape(N, D)
    u_pack = jnp.concatenate([U1, U2, U3], axis=0).T.astype(jnp.bfloat16)
    pwb = P_w.T.astype(jnp.bfloat16)

    grid = (N // _TM, _NL)

    hb, cc = pl.pallas_call(
        _layer1_body,
        grid=grid,
        in_specs=[
            pl.BlockSpec((_TM, D), lambda t, l: (t, 0)),
            pl.BlockSpec(pwb.shape, lambda t, l: (0, 0)),
            pl.BlockSpec(u_pack.shape, lambda t, l: (0, 0)),
            pl.BlockSpec((1, D, D),
                         lambda t, l: (jnp.where(t < 1, l, 0), 0, 0)),
        ],
        out_specs=[
            pl.BlockSpec((_TM, D), lambda t, l: (t, 0)),
            pl.BlockSpec((_TM, 16), lambda t, l: (t, 0)),
        ],
        out_shape=[
            jax.ShapeDtypeStruct((N, D), jnp.bfloat16),
            jax.ShapeDtypeStruct((N, 16), jnp.float32),
        ],
        scratch_shapes=[
            pltpu.VMEM((_TM, D), jnp.float32),
            pltpu.VMEM((_NL, D, D), jnp.bfloat16),
            pltpu.VMEM((_TM, _NL), jnp.float32),
        ],
    )(x_flat, pwb, u_pack, W1)

    y = pl.pallas_call(
        _layer2_body,
        grid=grid,
        in_specs=[
            pl.BlockSpec((_TM, D), lambda t, l: (t, 0)),
            pl.BlockSpec((_TM, 16), lambda t, l: (t, 0)),
            pl.BlockSpec(b2.shape, lambda t, l: (0, 0)),
            pl.BlockSpec((1, DO, D),
                         lambda t, l: (jnp.where(t < 1, l, 0), 0, 0)),
        ],
        out_specs=pl.BlockSpec((_TM, DO), lambda t, l: (t, 0)),
        out_shape=jax.ShapeDtypeStruct((N, DO), jnp.float32),
        scratch_shapes=[
            pltpu.VMEM((_TM, DO), jnp.float32),
            pltpu.VMEM((_NL, DO, D), jnp.bfloat16),
        ],
    )(hb, cc, b2, W2)
    return y.reshape(Bx, Tx, DO)


# speedup vs baseline: 1.3047x; 1.3047x over previous
import functools

import jax
import jax.numpy as jnp
from jax.experimental import pallas as pl
from jax.experimental.pallas import tpu as pltpu

_HIGH = jax.lax.Precision.HIGHEST


def _top2_coeff(z, tau):
    idx = jax.lax.broadcasted_iota(jnp.int32, z.shape, 1)
    v1 = jnp.max(z, axis=1, keepdims=True)
    i1 = jnp.min(jnp.where(z == v1, idx, z.shape[1]), axis=1, keepdims=True)
    m1 = idx == i1
    z2 = jnp.where(m1, -jnp.inf, z)
    v2 = jnp.max(z2, axis=1, keepdims=True)
    i2 = jnp.min(jnp.where(z2 == v2, idx, z.shape[1]), axis=1, keepdims=True)
    m2 = idx == i2
    t = tau + 1e-8
    a = jnp.exp((v2 - v1) / t)
    w1 = 1.0 / (1.0 + a)
    w2 = a / (1.0 + a)
    return jnp.where(m1, w1, 0.0) + jnp.where(m2, w2, 0.0)


def _fused_body(x_ref, pw_ref, u_ref, w1_ref, w2_ref, b2_ref, out_ref):
    xt = x_ref[...]
    xb = xt.astype(jnp.bfloat16)
    addr = jax.lax.dot_general(xb, pw_ref[...], (((1,), (0,)), ((), ())),
                               preferred_element_type=jnp.float32)
    zz = jax.lax.dot_general(addr.astype(jnp.bfloat16), u_ref[...],
                             (((1,), (0,)), ((), ())),
                             preferred_element_type=jnp.float32)
    c1 = _top2_coeff(zz[:, 0:8], 1.0)
    c2 = _top2_coeff(zz[:, 8:16], 1.0)
    c3 = _top2_coeff(zz[:, 16:24], 1.0)

    h = None
    for l in range(8):
        yl = jax.lax.dot_general(xb, w1_ref[l], (((1,), (1,)), ((), ())),
                                 preferred_element_type=jnp.float32)
        h = yl * c1[:, l:l + 1] if h is None else h + yl * c1[:, l:l + 1]
    h = 0.5 * h * (1.0 + jax.lax.erf(h * 0.7071067811865476))

    hb = h.astype(jnp.bfloat16)
    y = jax.lax.dot_general(c3, b2_ref[...], (((1,), (0,)), ((), ())),
                            precision=_HIGH,
                            preferred_element_type=jnp.float32)
    for l in range(8):
        yl = jax.lax.dot_general(hb, w2_ref[l], (((1,), (1,)), ((), ())),
                                 preferred_element_type=jnp.float32)
        y = y + yl * c2[:, l:l + 1]
    out_ref[...] = y


@functools.partial(jax.jit, static_argnames=())
def kernel(x, P_w, U1, U2, U3, W1, W2, b2):
    Bx, Tx, D = x.shape
    N = Bx * Tx
    H = W1.shape[1]
    DO = W2.shape[1]
    x_flat = x.reshape(N, D)
    u_pack = jnp.concatenate([U1, U2, U3], axis=0).T.astype(jnp.bfloat16)
    pwb = P_w.T.astype(jnp.bfloat16)
    w1b = W1.astype(jnp.bfloat16)
    w2b = W2.astype(jnp.bfloat16)

    TM = 512
    grid = (N // TM,)
    out = pl.pallas_call(
        _fused_body,
        grid=grid,
        in_specs=[
            pl.BlockSpec((TM, D), lambda i: (i, 0)),
            pl.BlockSpec(pwb.shape, lambda i: (0, 0)),
            pl.BlockSpec(u_pack.shape, lambda i: (0, 0)),
            pl.BlockSpec(w1b.shape, lambda i: (0, 0, 0)),
            pl.BlockSpec(w2b.shape, lambda i: (0, 0, 0)),
            pl.BlockSpec(b2.shape, lambda i: (0, 0)),
        ],
        out_specs=pl.BlockSpec((TM, DO), lambda i: (i, 0)),
        out_shape=jax.ShapeDtypeStruct((N, DO), jnp.float32),
    )(x_flat, pwb, u_pack, w1b, w2b, b2)
    return out.reshape(Bx, Tx, DO)


# speedup vs baseline: 1.4506x; 1.1118x over previous
import functools

import jax
import jax.numpy as jnp
from jax.experimental import pallas as pl
from jax.experimental.pallas import tpu as pltpu

_HIGH = jax.lax.Precision.HIGHEST
_NL = 8
_TM = 256


def _top2_coeff(z, tau):
    idx = jax.lax.broadcasted_iota(jnp.int32, z.shape, 1)
    v1 = jnp.max(z, axis=1, keepdims=True)
    i1 = jnp.min(jnp.where(z == v1, idx, z.shape[1]), axis=1, keepdims=True)
    m1 = idx == i1
    z2 = jnp.where(m1, -jnp.inf, z)
    v2 = jnp.max(z2, axis=1, keepdims=True)
    i2 = jnp.min(jnp.where(z2 == v2, idx, z.shape[1]), axis=1, keepdims=True)
    m2 = idx == i2
    t = tau + 1e-8
    a = jnp.exp((v2 - v1) / t)
    w1 = 1.0 / (1.0 + a)
    w2 = a / (1.0 + a)
    return jnp.where(m1, w1, 0.0) + jnp.where(m2, w2, 0.0)


def _layer1_body(x_ref, pw_ref, u_ref, w_ref, hb_ref, cc_ref, wb_ref):
    t = pl.program_id(0)

    @pl.when(t == 0)
    def _cast():
        for l in range(_NL):
            wb_ref[l] = w_ref[l].astype(jnp.bfloat16)

    xb = x_ref[...].astype(jnp.bfloat16)
    addr = jax.lax.dot_general(xb, pw_ref[...], (((1,), (0,)), ((), ())),
                               preferred_element_type=jnp.float32)
    zz = jax.lax.dot_general(addr.astype(jnp.bfloat16), u_ref[...],
                             (((1,), (0,)), ((), ())),
                             preferred_element_type=jnp.float32)
    c1 = _top2_coeff(zz[:, 0:8], 1.0)
    cc_ref[...] = jnp.concatenate(
        [_top2_coeff(zz[:, 8:16], 1.0), _top2_coeff(zz[:, 16:24], 1.0)],
        axis=1)

    h = None
    for l in range(_NL):
        yl = jax.lax.dot_general(xb, wb_ref[l], (((1,), (1,)), ((), ())),
                                 preferred_element_type=jnp.float32)
        h = yl * c1[:, l:l + 1] if h is None else h + yl * c1[:, l:l + 1]
    h = 0.5 * h * (1.0 + jax.lax.erf(h * 0.7071067811865476))
    hb_ref[...] = h.astype(jnp.bfloat16)


def _layer2_body(hb_ref, cc_ref, b2_ref, w_ref, y_ref, wb_ref):
    t = pl.program_id(0)

    @pl.when(t == 0)
    def _cast():
        for l in range(_NL):
            wb_ref[l] = w_ref[l].astype(jnp.bfloat16)

    hb = hb_ref[...]
    y = jax.lax.dot_general(cc_ref[:, 8:16], b2_ref[...],
                            (((1,), (0,)), ((), ())),
                            precision=_HIGH,
                            preferred_element_type=jnp.float32)
    for l in range(_NL):
        yl = jax.lax.dot_general(hb, wb_ref[l], (((1,), (1,)), ((), ())),
                                 preferred_element_type=jnp.float32)
        y = y + yl * cc_ref[:, l:l + 1]
    y_ref[...] = y


@functools.partial(jax.jit, static_argnames=())
def kernel(x, P_w, U1, U2, U3, W1, W2, b2):
    Bx, Tx, D = x.shape
    N = Bx * Tx
    DO = W2.shape[1]
    x_flat = x.reshape(N, D)
    u_pack = jnp.concatenate([U1, U2, U3], axis=0).T.astype(jnp.bfloat16)
    pwb = P_w.T.astype(jnp.bfloat16)

    grid = (N // _TM,)

    hb, cc = pl.pallas_call(
        _layer1_body,
        grid=grid,
        in_specs=[
            pl.BlockSpec((_TM, D), lambda t: (t, 0)),
            pl.BlockSpec(pwb.shape, lambda t: (0, 0)),
            pl.BlockSpec(u_pack.shape, lambda t: (0, 0)),
            pl.BlockSpec(W1.shape, lambda t: (0, 0, 0)),
        ],
        out_specs=[
            pl.BlockSpec((_TM, D), lambda t: (t, 0)),
            pl.BlockSpec((_TM, 16), lambda t: (t, 0)),
        ],
        out_shape=[
            jax.ShapeDtypeStruct((N, D), jnp.bfloat16),
            jax.ShapeDtypeStruct((N, 16), jnp.float32),
        ],
        scratch_shapes=[pltpu.VMEM((_NL, D, D), jnp.bfloat16)],
    )(x_flat, pwb, u_pack, W1)

    y = pl.pallas_call(
        _layer2_body,
        grid=grid,
        in_specs=[
            pl.BlockSpec((_TM, D), lambda t: (t, 0)),
            pl.BlockSpec((_TM, 16), lambda t: (t, 0)),
            pl.BlockSpec(b2.shape, lambda t: (0, 0)),
            pl.BlockSpec(W2.shape, lambda t: (0, 0, 0)),
        ],
        out_specs=pl.BlockSpec((_TM, DO), lambda t: (t, 0)),
        out_shape=jax.ShapeDtypeStruct((N, DO), jnp.float32),
        scratch_shapes=[pltpu.VMEM((_NL, DO, D), jnp.bfloat16)],
    )(hb, cc, b2, W2)
    return y.reshape(Bx, Tx, DO)
